# baseline (speedup 1.0000x reference)
import functools

import jax
import jax.numpy as jnp
from jax import lax
from jax.experimental import pallas as pl
from jax.experimental.pallas import tpu as pltpu
from jax.experimental.pallas import tpu_sc as plsc

HIDDEN = 16
NUM_WORKERS = 32
CHUNK = 3200


def _gather_body(ids_hbm, table_hbm, out_hbm,
                 idx_a, idx_b, rows_a, rows_b, sem_a, sem_b,
                 *, rows_per_worker, n_chunks):
    wid = lax.axis_index("s") * 2 + lax.axis_index("c")
    base = wid * rows_per_worker

    idx = (idx_a, idx_b)
    rows = (rows_a, rows_b)
    sems = (sem_a, sem_b)

    prev = None
    for j in range(n_chunks):
        s = j % 2
        off = base + j * CHUNK
        pltpu.sync_copy(ids_hbm.at[pl.ds(off, CHUNK)], idx[s])
        cp = pltpu.async_copy(table_hbm.at[idx[s]], rows[s], sems[s])
        if prev is not None:
            pcp, ps, poff = prev
            pcp.wait()
            pltpu.sync_copy(rows[ps],
                            out_hbm.at[pl.ds(poff, CHUNK), pl.ds(0, HIDDEN)])
        prev = (cp, s, off)
    pcp, ps, poff = prev
    pcp.wait()
    pltpu.sync_copy(rows[ps],
                    out_hbm.at[pl.ds(poff, CHUNK), pl.ds(0, HIDDEN)])


def kernel(input_ids, attention_mask, emb_weight):
    del attention_mask
    batch, seq = input_ids.shape
    vocab = emb_weight.shape[0]
    total = batch * seq
    rows_per_worker = total // NUM_WORKERS
    n_chunks = rows_per_worker // CHUNK

    flat_ids = input_ids.reshape(total).astype(jnp.int32)

    one = jnp.float32(1) - (flat_ids[0] >> 31).astype(jnp.float32)
    table = emb_weight * one

    mesh = plsc.VectorSubcoreMesh(core_axis_name="c", subcore_axis_name="s")
    out2d = pl.kernel(
        functools.partial(_gather_body, rows_per_worker=rows_per_worker,
                          n_chunks=n_chunks),
        out_type=jax.ShapeDtypeStruct((total, 128), jnp.float32),
        mesh=mesh,
        scratch_types=[
            pltpu.VMEM((CHUNK,), jnp.int32),
            pltpu.VMEM((CHUNK,), jnp.int32),
            pltpu.VMEM((CHUNK, HIDDEN), jnp.float32),
            pltpu.VMEM((CHUNK, HIDDEN), jnp.float32),
            pltpu.SemaphoreType.DMA,
            pltpu.SemaphoreType.DMA,
        ],
        compiler_params=pltpu.CompilerParams(use_tc_tiling_on_sc=False),
    )(flat_ids, table)

    return out2d[:, :HIDDEN].reshape(batch, seq, HIDDEN)


# speedup vs baseline: 1.4505x; 1.4505x over previous
import functools

import jax
import jax.numpy as jnp
from jax import lax
from jax.experimental import pallas as pl
from jax.experimental.pallas import tpu as pltpu
from jax.experimental.pallas import tpu_sc as plsc

HIDDEN = 16
NUM_WORKERS = 32
CHUNK = 3200


def _gather_body(ids_hbm, table_hbm, out_hbm,
                 idx_a, idx_b, rows_a, rows_b, sem_a, sem_b,
                 *, rows_per_worker, n_chunks):
    wid = lax.axis_index("s") * 2 + lax.axis_index("c")
    base = wid * rows_per_worker

    idx = (idx_a, idx_b)
    rows = (rows_a, rows_b)
    sems = (sem_a, sem_b)

    prev = None
    for j in range(n_chunks):
        s = j % 2
        off = base + j * CHUNK
        pltpu.sync_copy(ids_hbm.at[pl.ds(off, CHUNK)], idx[s])
        cp = pltpu.async_copy(table_hbm.at[idx[s]], rows[s], sems[s])
        if prev is not None:
            pcp, ps, poff = prev
            pcp.wait()
            pltpu.sync_copy(rows[ps],
                            out_hbm.at[pl.ds(poff, CHUNK), pl.ds(0, HIDDEN)])
        prev = (cp, s, off)
    pcp, ps, poff = prev
    pcp.wait()
    pltpu.sync_copy(rows[ps],
                    out_hbm.at[pl.ds(poff, CHUNK), pl.ds(0, HIDDEN)])


def kernel(input_ids, attention_mask, emb_weight):
    del attention_mask
    batch, seq = input_ids.shape
    vocab = emb_weight.shape[0]
    total = batch * seq
    rows_per_worker = total // NUM_WORKERS
    n_chunks = rows_per_worker // CHUNK

    flat_ids = input_ids.reshape(total).astype(jnp.int32)

    table = emb_weight

    mesh = plsc.VectorSubcoreMesh(core_axis_name="c", subcore_axis_name="s")
    out2d = pl.kernel(
        functools.partial(_gather_body, rows_per_worker=rows_per_worker,
                          n_chunks=n_chunks),
        out_type=jax.ShapeDtypeStruct((total, 128), jnp.float32),
        mesh=mesh,
        scratch_types=[
            pltpu.VMEM((CHUNK,), jnp.int32),
            pltpu.VMEM((CHUNK,), jnp.int32),
            pltpu.VMEM((CHUNK, HIDDEN), jnp.float32),
            pltpu.VMEM((CHUNK, HIDDEN), jnp.float32),
            pltpu.SemaphoreType.DMA,
            pltpu.SemaphoreType.DMA,
        ],
        compiler_params=pltpu.CompilerParams(use_tc_tiling_on_sc=False),
    )(flat_ids, table)

    return out2d.reshape(batch, seq, 128)[:, :, :HIDDEN]
